# Initial kernel scaffold; baseline (speedup 1.0000x reference)
import jax
import jax.numpy as jnp
from jax.experimental import pallas as pl


def kernel(features, nodes, neigh_idx, num_sample):
    raise NotImplementedError("write your pallas kernel here")


# speedup vs baseline: 1.1786x; 1.1786x over previous
import functools

import jax
import jax.numpy as jnp
from jax import lax
from jax.experimental import pallas as pl
from jax.experimental.pallas import tpu as pltpu
from jax.experimental.pallas import tpu_sc as plsc

_NC = 2
_NS = 16
_NW = _NC * _NS
_L = 16


def _build_kernel(B_pad, S, D, N, scale):
    C = 8
    bpw = B_pad // _NW
    n_chunks = bpw // C
    mesh = plsc.VectorSubcoreMesh(core_axis_name="c", subcore_axis_name="s")

    @functools.partial(
        pl.kernel,
        out_type=jax.ShapeDtypeStruct((B_pad, D), jnp.float32),
        mesh=mesh,
        scratch_types=[
            pltpu.VMEM((n_chunks, C * S), jnp.int32),
            pltpu.VMEM((C * S, D), jnp.float32),
            pltpu.VMEM((C, D), jnp.float32),
            pltpu.SemaphoreType.DMA,
        ],
    )
    def aggr(feat_hbm, nidx_hbm, out_hbm, idx_ref, g_ref, o_ref, sem):
        wid = lax.axis_index("s") * _NC + lax.axis_index("c")
        base_row = wid * bpw
        pltpu.sync_copy(nidx_hbm.at[wid], idx_ref)

        def chunk_body(j, carry):
            pltpu.async_copy(feat_hbm.at[idx_ref.at[j]], g_ref, sem).wait()

            def reduce_row(r, c2):
                row = r * S
                for v in range(D // _L):
                    sl = pl.ds(v * _L, _L)
                    acc = g_ref[row, sl]
                    for s in range(1, S):
                        acc = acc + g_ref[row + s, sl]
                    o_ref[r, sl] = acc * scale
                return c2

            lax.fori_loop(0, C, reduce_row, 0, unroll=False)
            pltpu.sync_copy(o_ref, out_hbm.at[pl.ds(base_row + j * C, C)])
            return carry

        lax.fori_loop(0, n_chunks, chunk_body, 0, unroll=False)

    return aggr


def kernel(features, nodes, neigh_idx, num_sample):
    N, D = features.shape
    B, S = neigh_idx.shape
    C = 8
    B_pad = ((B + C * _NW - 1) // (C * _NW)) * (C * _NW)
    nidx = neigh_idx.astype(jnp.int32)
    if B_pad != B:
        nidx = jnp.pad(nidx, ((0, B_pad - B), (0, 0)))
    nidx = nidx.reshape(_NW, (B_pad // _NW) // C, C * S)

    aggr = _build_kernel(B_pad, S, D, N, 1.0 / float(S))
    out = aggr(features, nidx)
    return out[:B]


# speedup vs baseline: 1.5593x; 1.3230x over previous
import functools

import jax
import jax.numpy as jnp
from jax import lax
from jax.experimental import pallas as pl
from jax.experimental.pallas import tpu as pltpu
from jax.experimental.pallas import tpu_sc as plsc

_NC = 2
_NS = 16
_NW = _NC * _NS
_L = 16

_C = 8


def _build_kernel(B_pad, S, D, scale):
    C = _C
    bpw = B_pad // _NW
    n_chunks = bpw // C
    mesh = plsc.VectorSubcoreMesh(core_axis_name="c", subcore_axis_name="s")

    @functools.partial(
        pl.kernel,
        out_type=jax.ShapeDtypeStruct((B_pad, D), jnp.float32),
        mesh=mesh,
        scratch_types=[
            pltpu.VMEM((n_chunks, C * S), jnp.int32),
            pltpu.VMEM((C * S, D), jnp.float32),
            pltpu.VMEM((C * S, D), jnp.float32),
            pltpu.VMEM((C, D), jnp.float32),
            pltpu.VMEM((C, D), jnp.float32),
            pltpu.SemaphoreType.DMA,
            pltpu.SemaphoreType.DMA,
            pltpu.SemaphoreType.DMA,
            pltpu.SemaphoreType.DMA,
        ],
    )
    def aggr(feat_hbm, nidx_hbm, out_hbm, idx_ref, g0, g1, o0, o1,
             sg0, sg1, so0, so1):
        wid = lax.axis_index("s") * _NC + lax.axis_index("c")
        base_row = wid * bpw
        pltpu.sync_copy(nidx_hbm.at[wid], idx_ref)

        bufs = ((g0, sg0, o0, so0), (g1, sg1, o1, so1))

        pltpu.async_copy(feat_hbm.at[idx_ref.at[0]], g0, sg0)
        pltpu.async_copy(feat_hbm.at[idx_ref.at[1]], g1, sg1)

        def pair_body(p, carry):
            j = p * 2
            for b, (g, sg, o, so) in enumerate(bufs):
                jj = j + b
                pltpu.make_async_copy(feat_hbm.at[idx_ref.at[jj]], g, sg).wait()

                @pl.when(p > 0)
                def _wait_out():
                    pltpu.make_async_copy(
                        o, out_hbm.at[pl.ds(base_row + (jj - 2) * C, C)], so
                    ).wait()

                def reduce_row(r, c2):
                    row = r * S
                    for v in range(D // _L):
                        sl = pl.ds(v * _L, _L)
                        acc = g[row, sl]
                        for s in range(1, S):
                            acc = acc + g[row + s, sl]
                        o[r, sl] = acc * scale
                    return c2

                lax.fori_loop(0, C, reduce_row, 0, unroll=False)
                pltpu.async_copy(
                    o, out_hbm.at[pl.ds(base_row + jj * C, C)], so)

                @pl.when(jj + 2 < n_chunks)
                def _next_gather():
                    pltpu.async_copy(feat_hbm.at[idx_ref.at[jj + 2]], g, sg)

            return carry

        lax.fori_loop(0, n_chunks // 2, pair_body, 0, unroll=False)
        pltpu.make_async_copy(
            o0, out_hbm.at[pl.ds(base_row + (n_chunks - 2) * C, C)], so0
        ).wait()
        pltpu.make_async_copy(
            o1, out_hbm.at[pl.ds(base_row + (n_chunks - 1) * C, C)], so1
        ).wait()

    return aggr


def kernel(features, nodes, neigh_idx, num_sample):
    N, D = features.shape
    B, S = neigh_idx.shape
    step = 2 * _C * _NW
    B_pad = ((B + step - 1) // step) * step
    nidx = neigh_idx.astype(jnp.int32)
    if B_pad != B:
        nidx = jnp.pad(nidx, ((0, B_pad - B), (0, 0)))
    nidx = nidx.reshape(_NW, (B_pad // _NW) // _C, _C * S)

    aggr = _build_kernel(B_pad, S, D, 1.0 / float(S))
    out = aggr(features, nidx)
    return out[:B]
